# Initial kernel scaffold; baseline (speedup 1.0000x reference)
import jax
import jax.numpy as jnp
from jax.experimental import pallas as pl


def kernel(x, edge_index, W, att_src, att_dst, bias, gamma, beta, attW, attb, fcW, fcb):
    raise NotImplementedError("write your pallas kernel here")


# speedup vs baseline: 11.7819x; 11.7819x over previous
import functools

import jax
import jax.numpy as jnp
from jax import lax
from jax.experimental import pallas as pl
from jax.experimental.pallas import tpu as pltpu
from jax.experimental.pallas import tpu_sc as plsc

N = 10000
E = 160000
DIN = 256
H = 8
C = 64
HC = H * C
DOUT = 256

NC = 2
NS = 16
LANES = 16
KCH = 128
NCHUNK = 84
EP = NS * NCHUNK * KCH
HPC = H // NC
ACC_ROWS = 10240
ROW_W = 80
BN = 1000
NB = N // BN


def _prep_body(x_ref, w_ref, asw_ref, adw_ref, h_ref, asrc_ref, adst_ref):
    hb = jnp.dot(x_ref[...], w_ref[0], preferred_element_type=jnp.float32)
    h_ref[...] = hb
    asrc_ref[...] = jnp.sum(hb * asw_ref[0], axis=1, keepdims=True)[None]
    adst_ref[...] = jnp.sum(hb * adw_ref[0], axis=1, keepdims=True)[None]


def _prep(x, W3, att_src3, att_dst3):
    return pl.pallas_call(
        _prep_body,
        grid=(NB, H),
        in_specs=[
            pl.BlockSpec((BN, DIN), lambda nb, hd: (nb, 0)),
            pl.BlockSpec((1, DIN, C), lambda nb, hd: (hd, 0, 0)),
            pl.BlockSpec((1, 1, C), lambda nb, hd: (hd, 0, 0)),
            pl.BlockSpec((1, 1, C), lambda nb, hd: (hd, 0, 0)),
        ],
        out_specs=[
            pl.BlockSpec((BN, C), lambda nb, hd: (hd * NB + nb, 0)),
            pl.BlockSpec((1, BN, 1), lambda nb, hd: (hd, nb, 0)),
            pl.BlockSpec((1, BN, 1), lambda nb, hd: (hd, nb, 0)),
        ],
        out_shape=[
            jax.ShapeDtypeStruct((H * N, C), jnp.float32),
            jax.ShapeDtypeStruct((H, N, 1), jnp.float32),
            jax.ShapeDtypeStruct((H, N, 1), jnp.float32),
        ],
    )(x, W3, att_src3, att_dst3)


def _edge_body(hflat, asrcT, adstT, srcp, dstp, out80,
               srcb, dstb, asrcb, adstb, eab, gidx, rows, stage, zbuf, acc):
    c = lax.axis_index("c")
    s = lax.axis_index("s")

    pltpu.sync_copy(srcp.at[s], srcb)
    pltpu.sync_copy(dstp.at[s], dstb)

    z16 = jnp.zeros((16,), jnp.float32)

    def zrow(r, _):
        for q in range(ROW_W // 16):
            zbuf[r, pl.ds(q * 16, 16)] = z16
        return 0
    lax.fori_loop(0, 16, zrow, 0)

    lane0 = lax.iota(jnp.int32, 16) == 0
    z16i = jnp.zeros((16,), jnp.int32)

    for hd_local in range(HPC):
        hd = c * HPC + hd_local
        pltpu.sync_copy(asrcT.at[hd], asrcb)
        pltpu.sync_copy(adstT.at[hd], adstb)
        for k in range(ACC_ROWS // NS // 16):
            pltpu.sync_copy(zbuf, acc.at[pl.ds(s * (ACC_ROWS // NS) + k * 16, 16)])
        plsc.subcore_barrier()

        def chunk(j, _):
            kmask = jnp.full((16,), 1023, jnp.int32)
            kten = jnp.full((16,), 10, jnp.int32)
            knm1 = jnp.full((16,), N - 1, jnp.int32)
            khdn = jnp.full((16,), hd * N, jnp.int32)
            kslope = jnp.full((16,), 0.2, jnp.float32)

            def vec(i, _):
                sv = srcb[j, pl.ds(i * 16, 16)]
                dv = dstb[j, pl.ds(i * 16, 16)]
                dc = jnp.minimum(dv, knm1)
                a1 = plsc.load_gather(asrcb, [sv])
                a2 = plsc.load_gather(adstb, [dc])
                al = a1 + a2
                al = jnp.maximum(al, al * kslope)
                ea = jnp.exp(al)
                eab[pl.ds(i * 16, 16)] = ea
                gidx[pl.ds(i * 16, 16)] = sv + khdn
                return 0
            lax.fori_loop(0, KCH // 16, vec, 0)

            pltpu.sync_copy(hflat.at[gidx], rows)

            def edge(e, _):
                eav = plsc.load_gather(eab, [z16i + e])
                for q in range(C // 16):
                    stage[e, pl.ds(q * 16, 16)] = (
                        rows[e, pl.ds(q * 16, 16)] * eav)
                stage[e, pl.ds(C, 16)] = jnp.where(lane0, eav, 0.0)
                return 0
            lax.fori_loop(0, KCH, edge, 0)

            pltpu.sync_copy(stage, acc.at[dstb.at[j]], add=True)
            return 0
        lax.fori_loop(0, NCHUNK, chunk, 0)

        plsc.subcore_barrier()
        pltpu.sync_copy(acc.at[pl.ds(s * (ACC_ROWS // NS), ACC_ROWS // NS)],
                        out80.at[hd].at[s])
        plsc.subcore_barrier()


def _edge(hflat, asrcT, adstT, srcp, dstp):
    mesh = plsc.VectorSubcoreMesh(core_axis_name="c", subcore_axis_name="s")
    return pl.kernel(
        _edge_body,
        out_type=jax.ShapeDtypeStruct((H, NS, ACC_ROWS // NS, ROW_W), jnp.float32),
        mesh=mesh,
        compiler_params=pltpu.CompilerParams(
            needs_layout_passes=False, use_tc_tiling_on_sc=False),
        scratch_types=[
            pltpu.VMEM((NCHUNK, KCH), jnp.int32),
            pltpu.VMEM((NCHUNK, KCH), jnp.int32),
            pltpu.VMEM((10240,), jnp.float32),
            pltpu.VMEM((10240,), jnp.float32),
            pltpu.VMEM((KCH,), jnp.float32),
            pltpu.VMEM((KCH,), jnp.int32),
            pltpu.VMEM((KCH, C), jnp.float32),
            pltpu.VMEM((KCH, ROW_W), jnp.float32),
            pltpu.VMEM((16, ROW_W), jnp.float32),
            pltpu.VMEM_SHARED((ACC_ROWS, ROW_W), jnp.float32),
        ],
    )(hflat, asrcT, adstT, srcp, dstp)


def _stats_body(acc_ref, bias_ref, y_ref, sums_ref):
    nb = pl.program_id(0)
    yall = []
    for hd in range(H):
        a = acc_ref[hd]
        yall.append(a[:, :C] / (a[:, C:C + 1] + jnp.float32(1e-16)))
    y = jnp.concatenate(yall, axis=1) + bias_ref[...]
    y_ref[...] = y

    @pl.when(nb == 0)
    def _():
        sums_ref[...] = jnp.zeros_like(sums_ref)

    sums_ref[0:1, :] += jnp.sum(y, axis=0, keepdims=True)
    sums_ref[1:2, :] += jnp.sum(y * y, axis=0, keepdims=True)


def _stats(acc, bias2):
    return pl.pallas_call(
        _stats_body,
        grid=(NB,),
        in_specs=[
            pl.BlockSpec((H, BN, ROW_W), lambda nb: (0, nb, 0)),
            pl.BlockSpec((1, HC), lambda nb: (0, 0)),
        ],
        out_specs=[
            pl.BlockSpec((BN, HC), lambda nb: (nb, 0)),
            pl.BlockSpec((8, HC), lambda nb: (0, 0)),
        ],
        out_shape=[
            jax.ShapeDtypeStruct((N, HC), jnp.float32),
            jax.ShapeDtypeStruct((8, HC), jnp.float32),
        ],
    )(acc, bias2)


def _final_body(y_ref, sums_ref, gamma_ref, beta_ref, fcw_ref, fcb_ref, out_ref):
    s = sums_ref[...]
    mean = s[0:1, :] * jnp.float32(1.0 / N)
    var = s[1:2, :] * jnp.float32(1.0 / N) - mean * mean
    scale = gamma_ref[...] * lax.rsqrt(var + jnp.float32(1e-5))
    yn = (y_ref[...] - mean) * scale + beta_ref[...]
    z = jnp.where(yn > 0, yn, jnp.exp(jnp.minimum(yn, 0)) - jnp.float32(1.0))
    out_ref[...] = (jnp.dot(z, fcw_ref[...], preferred_element_type=jnp.float32)
                    + fcb_ref[...])


def _final(y, sums, gamma2, beta2, fcW, fcb2):
    return pl.pallas_call(
        _final_body,
        grid=(NB,),
        in_specs=[
            pl.BlockSpec((BN, HC), lambda nb: (nb, 0)),
            pl.BlockSpec((8, HC), lambda nb: (0, 0)),
            pl.BlockSpec((1, HC), lambda nb: (0, 0)),
            pl.BlockSpec((1, HC), lambda nb: (0, 0)),
            pl.BlockSpec((HC, DOUT), lambda nb: (0, 0)),
            pl.BlockSpec((1, DOUT), lambda nb: (0, 0)),
        ],
        out_specs=pl.BlockSpec((BN, DOUT), lambda nb: (nb, 0)),
        out_shape=jax.ShapeDtypeStruct((N, DOUT), jnp.float32),
    )(y, sums, gamma2, beta2, fcW, fcb2)


@jax.jit
def kernel(x, edge_index, W, att_src, att_dst, bias, gamma, beta, attW, attb, fcW, fcb):
    del attW, attb
    ei = edge_index.astype(jnp.int32)
    ar = jnp.arange(N, dtype=jnp.int32)
    src = jnp.concatenate([ei[0], ar])
    dst = jnp.concatenate([ei[1], ar])
    pad = EP - (E + N)
    srcp = jnp.concatenate([src, jnp.zeros((pad,), jnp.int32)]).reshape(NS, NCHUNK, KCH)
    dstp = jnp.concatenate([dst, jnp.full((pad,), N, jnp.int32)]).reshape(NS, NCHUNK, KCH)

    W3 = W.reshape(DIN, H, C).transpose(1, 0, 2)
    hflat, asrc, adst = _prep(x, W3, att_src.reshape(H, 1, C),
                              att_dst.reshape(H, 1, C))
    zpad = jnp.zeros((H, 240, 1), jnp.float32)
    asrcT = jnp.concatenate([asrc, zpad], axis=1).reshape(H, 10240)
    adstT = jnp.concatenate([adst, zpad], axis=1).reshape(H, 10240)

    acc = _edge(hflat, asrcT, adstT, srcp, dstp).reshape(H, ACC_ROWS, ROW_W)

    y, sums = _stats(acc, bias.reshape(1, HC))
    out = _final(y, sums, gamma.reshape(1, HC), beta.reshape(1, HC),
                 fcW, fcb.reshape(1, DOUT))
    return out


# speedup vs baseline: 16.4157x; 1.3933x over previous
import functools

import jax
import jax.numpy as jnp
from jax import lax
from jax.experimental import pallas as pl
from jax.experimental.pallas import tpu as pltpu
from jax.experimental.pallas import tpu_sc as plsc

N = 10000
E = 160000
DIN = 256
H = 8
C = 64
HC = H * C
DOUT = 256

NC = 2
NS = 16
LANES = 16
KCH = 128
NCHUNK = 84
EP = NS * NCHUNK * KCH
HPC = H // NC
ACC_ROWS = 10240
ROW_W = 80
ZROWS = 128
BN = 1000
NB = N // BN


def _prep_body(x_ref, w_ref, asw_ref, adw_ref, h_ref, asrc_ref, adst_ref):
    hb = jnp.dot(x_ref[...], w_ref[0], preferred_element_type=jnp.float32)
    h_ref[...] = hb
    asrc_ref[...] = jnp.sum(hb * asw_ref[0], axis=1, keepdims=True)[None]
    adst_ref[...] = jnp.sum(hb * adw_ref[0], axis=1, keepdims=True)[None]


def _prep(x, W3, att_src3, att_dst3):
    return pl.pallas_call(
        _prep_body,
        grid=(NB, H),
        in_specs=[
            pl.BlockSpec((BN, DIN), lambda nb, hd: (nb, 0)),
            pl.BlockSpec((1, DIN, C), lambda nb, hd: (hd, 0, 0)),
            pl.BlockSpec((1, 1, C), lambda nb, hd: (hd, 0, 0)),
            pl.BlockSpec((1, 1, C), lambda nb, hd: (hd, 0, 0)),
        ],
        out_specs=[
            pl.BlockSpec((BN, C), lambda nb, hd: (hd * NB + nb, 0)),
            pl.BlockSpec((1, BN, 1), lambda nb, hd: (hd, nb, 0)),
            pl.BlockSpec((1, BN, 1), lambda nb, hd: (hd, nb, 0)),
        ],
        out_shape=[
            jax.ShapeDtypeStruct((H * N, C), jnp.float32),
            jax.ShapeDtypeStruct((H, N, 1), jnp.float32),
            jax.ShapeDtypeStruct((H, N, 1), jnp.float32),
        ],
    )(x, W3, att_src3, att_dst3)


def _edge_body(hflat, asrcT, adstT, sdp, out80,
               asrcb, adstb, sdj2, dsti4, eab2, gidx2, rows2, stage2, zbuf,
               acc, sg0, sg1, sa0, sa1, sd0, sd1):
    c = lax.axis_index("c")
    s = lax.axis_index("s")

    z16 = jnp.zeros((16,), jnp.float32)

    def zrow(r, _):
        for q in range(ROW_W // 16):
            zbuf[r, pl.ds(q * 16, 16)] = z16
        return 0
    lax.fori_loop(0, ZROWS, zrow, 0)

    lane0 = lax.iota(jnp.int32, 16) == 0
    z16i = jnp.zeros((16,), jnp.int32)
    knm1 = jnp.full((16,), N - 1, jnp.int32)
    kslope = jnp.full((16,), 0.2, jnp.float32)
    kb = [jnp.zeros((16,), jnp.int32), jnp.full((16,), 1, jnp.int32)]
    sg = [sg0, sg1]
    sa = [sa0, sa1]
    sd = [sd0, sd1]
    jbase = s * NCHUNK

    def d_start(j, b):
        pltpu.async_copy(sdp.at[jbase + j], sdj2.at[b], sd[b])

    def d_wait(b):
        pltpu.make_async_copy(sdp.at[jbase], sdj2.at[b], sd[b]).wait()

    for hd_local in range(HPC):
        hd = c * HPC + hd_local
        khdn = jnp.full((16,), hd * N, jnp.int32)
        pltpu.sync_copy(asrcT.at[hd], asrcb)
        pltpu.sync_copy(adstT.at[hd], adstb)
        for k in range(ACC_ROWS // NS // ZROWS):
            pltpu.sync_copy(
                zbuf, acc.at[pl.ds(s * (ACC_ROWS // NS) + k * ZROWS, ZROWS)])
        plsc.subcore_barrier()

        def v_compute(j, b):
            def vec(i, _):
                sv = sdj2[b, 0, pl.ds(i * 16, 16)]
                dv = sdj2[b, 1, pl.ds(i * 16, 16)]
                dc = jnp.minimum(dv, knm1)
                al = plsc.load_gather(asrcb, [sv]) + plsc.load_gather(adstb, [dc])
                al = jnp.maximum(al, al * kslope)
                eab2[b, pl.ds(i * 16, 16)] = jnp.exp(al)
                gidx2[b, pl.ds(i * 16, 16)] = sv + khdn
                dsti4[j & 3, pl.ds(i * 16, 16)] = dv
                return 0
            lax.fori_loop(0, KCH // 16, vec, 0)

        def g_start(b):
            pltpu.async_copy(hflat.at[gidx2.at[b]], rows2.at[b], sg[b])

        def g_wait(b):
            pltpu.make_async_copy(hflat.at[gidx2.at[b]], rows2.at[b],
                                  sg[b]).wait()

        def s_scale(b):
            def edge(e, _):
                eav = plsc.load_gather(eab2, [kb[b], z16i + e])
                for q in range(C // 16):
                    stage2[b, e, pl.ds(q * 16, 16)] = (
                        rows2[b, e, pl.ds(q * 16, 16)] * eav)
                stage2[b, e, pl.ds(C, 16)] = jnp.where(lane0, eav, 0.0)
                return 0
            lax.fori_loop(0, KCH, edge, 0)

        def a_start(j, b):
            pltpu.async_copy(stage2.at[b], acc.at[dsti4.at[j & 3]], sa[b],
                             add=True)

        def a_wait(j, b):
            pltpu.make_async_copy(stage2.at[b], acc.at[dsti4.at[j & 3]],
                                  sa[b]).wait()

        d_start(0, 0)
        d_start(1, 1)
        d_wait(0)
        v_compute(0, 0)
        g_start(0)

        def body(t, _):
            j0 = 2 * t
            j1 = j0 + 1

            @pl.when(j0 + 2 < NCHUNK)
            def _():
                d_start(j0 + 2, 0)
            d_wait(1)

            @pl.when(t > 0)
            def _():
                a_wait(j1, 1)
            v_compute(j1, 1)
            g_start(1)

            @pl.when(j1 + 2 < NCHUNK)
            def _():
                d_start(j1 + 2, 1)
            g_wait(0)

            @pl.when(t > 0)
            def _():
                a_wait(j0, 0)
            s_scale(0)
            a_start(j0, 0)

            @pl.when(j0 + 2 < NCHUNK)
            def _():
                d_wait(0)
                v_compute(j0 + 2, 0)
                g_start(0)
            g_wait(1)
            s_scale(1)
            a_start(j1, 1)
            return 0
        lax.fori_loop(0, NCHUNK // 2, body, 0)
        a_wait(0, 0)
        a_wait(1, 1)

        plsc.subcore_barrier()
        pltpu.sync_copy(acc.at[pl.ds(s * (ACC_ROWS // NS), ACC_ROWS // NS)],
                        out80.at[hd].at[s])
        plsc.subcore_barrier()


def _edge(hflat, asrcT, adstT, sdp):
    mesh = plsc.VectorSubcoreMesh(core_axis_name="c", subcore_axis_name="s")
    return pl.kernel(
        _edge_body,
        out_type=jax.ShapeDtypeStruct((H, NS, ACC_ROWS // NS, ROW_W), jnp.float32),
        mesh=mesh,
        compiler_params=pltpu.CompilerParams(
            needs_layout_passes=False, use_tc_tiling_on_sc=False),
        scratch_types=[
            pltpu.VMEM((10240,), jnp.float32),
            pltpu.VMEM((10240,), jnp.float32),
            pltpu.VMEM((2, 2, KCH), jnp.int32),
            pltpu.VMEM((4, KCH), jnp.int32),
            pltpu.VMEM((2, KCH), jnp.float32),
            pltpu.VMEM((2, KCH), jnp.int32),
            pltpu.VMEM((2, KCH, C), jnp.float32),
            pltpu.VMEM((2, KCH, ROW_W), jnp.float32),
            pltpu.VMEM((ZROWS, ROW_W), jnp.float32),
            pltpu.VMEM_SHARED((ACC_ROWS, ROW_W), jnp.float32),
            pltpu.SemaphoreType.DMA,
            pltpu.SemaphoreType.DMA,
            pltpu.SemaphoreType.DMA,
            pltpu.SemaphoreType.DMA,
            pltpu.SemaphoreType.DMA,
            pltpu.SemaphoreType.DMA,
        ],
    )(hflat, asrcT, adstT, sdp)


def _stats_body(acc_ref, bias_ref, y_ref, sums_ref):
    nb = pl.program_id(0)
    yall = []
    for hd in range(H):
        a = acc_ref[hd]
        yall.append(a[:, :C] / (a[:, C:C + 1] + jnp.float32(1e-16)))
    y = jnp.concatenate(yall, axis=1) + bias_ref[...]
    y_ref[...] = y

    @pl.when(nb == 0)
    def _():
        sums_ref[...] = jnp.zeros_like(sums_ref)

    sums_ref[0:1, :] += jnp.sum(y, axis=0, keepdims=True)
    sums_ref[1:2, :] += jnp.sum(y * y, axis=0, keepdims=True)


def _stats(acc, bias2):
    return pl.pallas_call(
        _stats_body,
        grid=(NB,),
        in_specs=[
            pl.BlockSpec((H, BN, ROW_W), lambda nb: (0, nb, 0)),
            pl.BlockSpec((1, HC), lambda nb: (0, 0)),
        ],
        out_specs=[
            pl.BlockSpec((BN, HC), lambda nb: (nb, 0)),
            pl.BlockSpec((8, HC), lambda nb: (0, 0)),
        ],
        out_shape=[
            jax.ShapeDtypeStruct((N, HC), jnp.float32),
            jax.ShapeDtypeStruct((8, HC), jnp.float32),
        ],
    )(acc, bias2)


def _final_body(y_ref, sums_ref, gamma_ref, beta_ref, fcw_ref, fcb_ref, out_ref):
    s = sums_ref[...]
    mean = s[0:1, :] * jnp.float32(1.0 / N)
    var = s[1:2, :] * jnp.float32(1.0 / N) - mean * mean
    scale = gamma_ref[...] * lax.rsqrt(var + jnp.float32(1e-5))
    yn = (y_ref[...] - mean) * scale + beta_ref[...]
    z = jnp.where(yn > 0, yn, jnp.exp(jnp.minimum(yn, 0)) - jnp.float32(1.0))
    out_ref[...] = (jnp.dot(z, fcw_ref[...], preferred_element_type=jnp.float32)
                    + fcb_ref[...])


def _final(y, sums, gamma2, beta2, fcW, fcb2):
    return pl.pallas_call(
        _final_body,
        grid=(NB,),
        in_specs=[
            pl.BlockSpec((BN, HC), lambda nb: (nb, 0)),
            pl.BlockSpec((8, HC), lambda nb: (0, 0)),
            pl.BlockSpec((1, HC), lambda nb: (0, 0)),
            pl.BlockSpec((1, HC), lambda nb: (0, 0)),
            pl.BlockSpec((HC, DOUT), lambda nb: (0, 0)),
            pl.BlockSpec((1, DOUT), lambda nb: (0, 0)),
        ],
        out_specs=pl.BlockSpec((BN, DOUT), lambda nb: (nb, 0)),
        out_shape=jax.ShapeDtypeStruct((N, DOUT), jnp.float32),
    )(y, sums, gamma2, beta2, fcW, fcb2)


@jax.jit
def kernel(x, edge_index, W, att_src, att_dst, bias, gamma, beta, attW, attb, fcW, fcb):
    del attW, attb
    ei = edge_index.astype(jnp.int32)
    ar = jnp.arange(N, dtype=jnp.int32)
    src = jnp.concatenate([ei[0], ar])
    dst = jnp.concatenate([ei[1], ar])
    pad = EP - (E + N)
    srcp = jnp.concatenate([src, jnp.zeros((pad,), jnp.int32)]).reshape(NS, NCHUNK, KCH)
    dstp = jnp.concatenate([dst, jnp.full((pad,), N, jnp.int32)]).reshape(NS, NCHUNK, KCH)
    sdp = jnp.stack([srcp, dstp], axis=2).reshape(NS * NCHUNK, 2, KCH)

    W3 = W.reshape(DIN, H, C).transpose(1, 0, 2)
    hflat, asrc, adst = _prep(x, W3, att_src.reshape(H, 1, C),
                              att_dst.reshape(H, 1, C))
    zpad = jnp.zeros((H, 240, 1), jnp.float32)
    asrcT = jnp.concatenate([asrc, zpad], axis=1).reshape(H, 10240)
    adstT = jnp.concatenate([adst, zpad], axis=1).reshape(H, 10240)

    acc = _edge(hflat, asrcT, adstT, sdp).reshape(H, ACC_ROWS, ROW_W)

    y, sums = _stats(acc, bias.reshape(1, HC))
    out = _final(y, sums, gamma.reshape(1, HC), beta.reshape(1, HC),
                 fcW, fcb.reshape(1, DOUT))
    return out


# speedup vs baseline: 27.0456x; 1.6475x over previous
import functools

import jax
import jax.numpy as jnp
from jax import lax
from jax.experimental import pallas as pl
from jax.experimental.pallas import tpu as pltpu
from jax.experimental.pallas import tpu_sc as plsc

N = 10000
E = 160000
DIN = 256
H = 8
C = 64
HC = H * C
DOUT = 256

NC = 2
NS = 16
LANES = 16
KCH = 128
NCHUNK = 84
EP = NS * NCHUNK * KCH
HPC = H // NC
ACC_ROWS = 10240
ROW_W = 80
ZROWS = 128
BN = 1000
NB = N // BN


def _prep_body(x_ref, w_ref, asw_ref, adw_ref, h_ref, asrc_ref, adst_ref):
    hb = jnp.dot(x_ref[...], w_ref[0], preferred_element_type=jnp.float32)
    h_ref[...] = hb
    asrc_ref[...] = jnp.sum(hb * asw_ref[0], axis=1, keepdims=True)[None]
    adst_ref[...] = jnp.sum(hb * adw_ref[0], axis=1, keepdims=True)[None]


def _prep(x, W3, att_src3, att_dst3):
    return pl.pallas_call(
        _prep_body,
        grid=(NB, H),
        in_specs=[
            pl.BlockSpec((BN, DIN), lambda nb, hd: (nb, 0)),
            pl.BlockSpec((1, DIN, C), lambda nb, hd: (hd, 0, 0)),
            pl.BlockSpec((1, 1, C), lambda nb, hd: (hd, 0, 0)),
            pl.BlockSpec((1, 1, C), lambda nb, hd: (hd, 0, 0)),
        ],
        out_specs=[
            pl.BlockSpec((BN, C), lambda nb, hd: (hd * NB + nb, 0)),
            pl.BlockSpec((1, BN, 1), lambda nb, hd: (hd, nb, 0)),
            pl.BlockSpec((1, BN, 1), lambda nb, hd: (hd, nb, 0)),
        ],
        out_shape=[
            jax.ShapeDtypeStruct((H * N, C), jnp.float32),
            jax.ShapeDtypeStruct((H, N, 1), jnp.float32),
            jax.ShapeDtypeStruct((H, N, 1), jnp.float32),
        ],
    )(x, W3, att_src3, att_dst3)


def _edge_body(hflat, asrcT, adstT, sdp, out80,
               asrcb, adstb, sdj2, dsti4, eab2, gidx2, rows2, stage2, zbuf,
               acc, sg0, sg1, sa0, sa1, sd0, sd1):
    c = lax.axis_index("c")
    s = lax.axis_index("s")

    z16 = jnp.zeros((16,), jnp.float32)

    def zrow(r, _):
        for q in range(ROW_W // 16):
            zbuf[r, pl.ds(q * 16, 16)] = z16
        return 0
    lax.fori_loop(0, ZROWS, zrow, 0)

    lane0 = lax.iota(jnp.int32, 16) == 0
    z16i = jnp.zeros((16,), jnp.int32)
    knm1 = jnp.full((16,), N - 1, jnp.int32)
    kslope = jnp.full((16,), 0.2, jnp.float32)
    kb = [jnp.zeros((16,), jnp.int32), jnp.full((16,), 1, jnp.int32)]
    sg = [sg0, sg1]
    sa = [sa0, sa1]
    sd = [sd0, sd1]
    jbase = s * NCHUNK

    def d_start(j, b):
        pltpu.async_copy(sdp.at[jbase + j], sdj2.at[b], sd[b])

    def d_wait(b):
        pltpu.make_async_copy(sdp.at[jbase], sdj2.at[b], sd[b]).wait()

    for hd_local in range(HPC):
        hd = c * HPC + hd_local
        khdn = jnp.full((16,), hd * N, jnp.int32)
        pltpu.sync_copy(asrcT.at[hd], asrcb)
        pltpu.sync_copy(adstT.at[hd], adstb)
        for k in range(ACC_ROWS // NS // ZROWS):
            pltpu.sync_copy(
                zbuf, acc.at[pl.ds(s * (ACC_ROWS // NS) + k * ZROWS, ZROWS)])
        plsc.subcore_barrier()

        def v_compute(j, b):
            @plsc.parallel_loop(0, KCH, step=16, unroll=4)
            def _(i):
                sv = sdj2[b, 0, pl.ds(i, 16)]
                dv = sdj2[b, 1, pl.ds(i, 16)]
                dc = jnp.minimum(dv, knm1)
                al = plsc.load_gather(asrcb, [sv]) + plsc.load_gather(adstb, [dc])
                al = jnp.maximum(al, al * kslope)
                eab2[b, pl.ds(i, 16)] = jnp.exp(al)
                gidx2[b, pl.ds(i, 16)] = sv + khdn
                dsti4[j & 3, pl.ds(i, 16)] = dv

        def g_start(b):
            pltpu.async_copy(hflat.at[gidx2.at[b]], rows2.at[b], sg[b])

        def g_wait(b):
            pltpu.make_async_copy(hflat.at[gidx2.at[b]], rows2.at[b],
                                  sg[b]).wait()

        def s_scale(b):
            @plsc.parallel_loop(0, KCH, step=1, unroll=8)
            def _(e):
                eav = plsc.load_gather(eab2, [kb[b], z16i + e])
                for q in range(C // 16):
                    stage2[b, e, pl.ds(q * 16, 16)] = (
                        rows2[b, e, pl.ds(q * 16, 16)] * eav)
                stage2[b, e, pl.ds(C, 16)] = jnp.where(lane0, eav, 0.0)

        def a_start(j, b):
            pltpu.async_copy(stage2.at[b], acc.at[dsti4.at[j & 3]], sa[b],
                             add=True)

        def a_wait(j, b):
            pltpu.make_async_copy(stage2.at[b], acc.at[dsti4.at[j & 3]],
                                  sa[b]).wait()

        d_start(0, 0)
        d_start(1, 1)
        d_wait(0)
        v_compute(0, 0)
        g_start(0)

        def body(t, _):
            j0 = 2 * t
            j1 = j0 + 1

            @pl.when(j0 + 2 < NCHUNK)
            def _():
                d_start(j0 + 2, 0)
            d_wait(1)

            @pl.when(t > 0)
            def _():
                a_wait(j1, 1)
            v_compute(j1, 1)
            g_start(1)

            @pl.when(j1 + 2 < NCHUNK)
            def _():
                d_start(j1 + 2, 1)
            g_wait(0)

            @pl.when(t > 0)
            def _():
                a_wait(j0, 0)
            s_scale(0)
            a_start(j0, 0)

            @pl.when(j0 + 2 < NCHUNK)
            def _():
                d_wait(0)
                v_compute(j0 + 2, 0)
                g_start(0)
            g_wait(1)
            s_scale(1)
            a_start(j1, 1)
            return 0
        lax.fori_loop(0, NCHUNK // 2, body, 0)
        a_wait(0, 0)
        a_wait(1, 1)

        plsc.subcore_barrier()
        pltpu.sync_copy(acc.at[pl.ds(s * (ACC_ROWS // NS), ACC_ROWS // NS)],
                        out80.at[hd].at[s])
        plsc.subcore_barrier()


def _edge(hflat, asrcT, adstT, sdp):
    mesh = plsc.VectorSubcoreMesh(core_axis_name="c", subcore_axis_name="s")
    return pl.kernel(
        _edge_body,
        out_type=jax.ShapeDtypeStruct((H, NS, ACC_ROWS // NS, ROW_W), jnp.float32),
        mesh=mesh,
        compiler_params=pltpu.CompilerParams(
            needs_layout_passes=False, use_tc_tiling_on_sc=False),
        scratch_types=[
            pltpu.VMEM((10240,), jnp.float32),
            pltpu.VMEM((10240,), jnp.float32),
            pltpu.VMEM((2, 2, KCH), jnp.int32),
            pltpu.VMEM((4, KCH), jnp.int32),
            pltpu.VMEM((2, KCH), jnp.float32),
            pltpu.VMEM((2, KCH), jnp.int32),
            pltpu.VMEM((2, KCH, C), jnp.float32),
            pltpu.VMEM((2, KCH, ROW_W), jnp.float32),
            pltpu.VMEM((ZROWS, ROW_W), jnp.float32),
            pltpu.VMEM_SHARED((ACC_ROWS, ROW_W), jnp.float32),
            pltpu.SemaphoreType.DMA,
            pltpu.SemaphoreType.DMA,
            pltpu.SemaphoreType.DMA,
            pltpu.SemaphoreType.DMA,
            pltpu.SemaphoreType.DMA,
            pltpu.SemaphoreType.DMA,
        ],
    )(hflat, asrcT, adstT, sdp)


def _stats_body(acc_ref, bias_ref, y_ref, sums_ref):
    nb = pl.program_id(0)
    yall = []
    for hd in range(H):
        a = acc_ref[hd]
        yall.append(a[:, :C] / (a[:, C:C + 1] + jnp.float32(1e-16)))
    y = jnp.concatenate(yall, axis=1) + bias_ref[...]
    y_ref[...] = y

    @pl.when(nb == 0)
    def _():
        sums_ref[...] = jnp.zeros_like(sums_ref)

    sums_ref[0:1, :] += jnp.sum(y, axis=0, keepdims=True)
    sums_ref[1:2, :] += jnp.sum(y * y, axis=0, keepdims=True)


def _stats(acc, bias2):
    return pl.pallas_call(
        _stats_body,
        grid=(NB,),
        in_specs=[
            pl.BlockSpec((H, BN, ROW_W), lambda nb: (0, nb, 0)),
            pl.BlockSpec((1, HC), lambda nb: (0, 0)),
        ],
        out_specs=[
            pl.BlockSpec((BN, HC), lambda nb: (nb, 0)),
            pl.BlockSpec((8, HC), lambda nb: (0, 0)),
        ],
        out_shape=[
            jax.ShapeDtypeStruct((N, HC), jnp.float32),
            jax.ShapeDtypeStruct((8, HC), jnp.float32),
        ],
    )(acc, bias2)


def _final_body(y_ref, sums_ref, gamma_ref, beta_ref, fcw_ref, fcb_ref, out_ref):
    s = sums_ref[...]
    mean = s[0:1, :] * jnp.float32(1.0 / N)
    var = s[1:2, :] * jnp.float32(1.0 / N) - mean * mean
    scale = gamma_ref[...] * lax.rsqrt(var + jnp.float32(1e-5))
    yn = (y_ref[...] - mean) * scale + beta_ref[...]
    z = jnp.where(yn > 0, yn, jnp.exp(jnp.minimum(yn, 0)) - jnp.float32(1.0))
    out_ref[...] = (jnp.dot(z, fcw_ref[...], preferred_element_type=jnp.float32)
                    + fcb_ref[...])


def _final(y, sums, gamma2, beta2, fcW, fcb2):
    return pl.pallas_call(
        _final_body,
        grid=(NB,),
        in_specs=[
            pl.BlockSpec((BN, HC), lambda nb: (nb, 0)),
            pl.BlockSpec((8, HC), lambda nb: (0, 0)),
            pl.BlockSpec((1, HC), lambda nb: (0, 0)),
            pl.BlockSpec((1, HC), lambda nb: (0, 0)),
            pl.BlockSpec((HC, DOUT), lambda nb: (0, 0)),
            pl.BlockSpec((1, DOUT), lambda nb: (0, 0)),
        ],
        out_specs=pl.BlockSpec((BN, DOUT), lambda nb: (nb, 0)),
        out_shape=jax.ShapeDtypeStruct((N, DOUT), jnp.float32),
    )(y, sums, gamma2, beta2, fcW, fcb2)


@jax.jit
def kernel(x, edge_index, W, att_src, att_dst, bias, gamma, beta, attW, attb, fcW, fcb):
    del attW, attb
    ei = edge_index.astype(jnp.int32)
    ar = jnp.arange(N, dtype=jnp.int32)
    src = jnp.concatenate([ei[0], ar])
    dst = jnp.concatenate([ei[1], ar])
    pad = EP - (E + N)
    srcp = jnp.concatenate([src, jnp.zeros((pad,), jnp.int32)]).reshape(NS, NCHUNK, KCH)
    dstp = jnp.concatenate([dst, jnp.full((pad,), N, jnp.int32)]).reshape(NS, NCHUNK, KCH)
    sdp = jnp.stack([srcp, dstp], axis=2).reshape(NS * NCHUNK, 2, KCH)

    W3 = W.reshape(DIN, H, C).transpose(1, 0, 2)
    hflat, asrc, adst = _prep(x, W3, att_src.reshape(H, 1, C),
                              att_dst.reshape(H, 1, C))
    zpad = jnp.zeros((H, 240, 1), jnp.float32)
    asrcT = jnp.concatenate([asrc, zpad], axis=1).reshape(H, 10240)
    adstT = jnp.concatenate([adst, zpad], axis=1).reshape(H, 10240)

    acc = _edge(hflat, asrcT, adstT, sdp).reshape(H, ACC_ROWS, ROW_W)

    y, sums = _stats(acc, bias.reshape(1, HC))
    out = _final(y, sums, gamma.reshape(1, HC), beta.reshape(1, HC),
                 fcW, fcb.reshape(1, DOUT))
    return out
